# baseline (speedup 1.0000x reference)
import jax
import jax.numpy as jnp
from jax import lax
from jax.experimental import pallas as pl

B = 625
A = 16
N = B * A
DIN = 128
H1 = 64
DP = 64
WOUT = 64
ACT = 8

TB = 625
NB = (B + TB - 1) // TB

_DN11 = (((1,), (1,)), ((), ()))


def _critic_kernel(x3_ref, pi3_ref, ac3_ref,
                   w1_ref, b1_ref, w2_ref, b2_ref, wfc_ref,
                   wat_ref, wv_ref, bv_ref,
                   xv_ref, wmb_ref):
    f32 = jnp.float32
    xm = jnp.sum(x3_ref[...], axis=1) * (1.0 / A)

    h = jnp.maximum(
        lax.dot_general(xm, w1_ref[...], _DN11, preferred_element_type=f32)
        + b1_ref[...], 0.0)
    o = (lax.dot_general(h, w2_ref[...], _DN11, preferred_element_type=f32)
         + b2_ref[...])
    zl = lax.dot_general(o, wfc_ref[...], _DN11, preferred_element_type=f32)

    wa = wat_ref[:, :WOUT] + wat_ref[:, WOUT:]
    e = jnp.sum(zl * wa, axis=1, keepdims=True)
    e = jnp.where(e >= 0.0, e, 0.01 * e)
    w = jax.nn.sigmoid(e)

    wz3 = wv_ref[:, DP:].reshape(1, 1, ACT)
    pi3 = pi3_ref[...]
    d3 = pi3 - ac3_ref[...]
    gj = jnp.sum(d3 * wz3, axis=2)
    pj = jnp.sum(pi3 * wz3, axis=2)
    sp = jnp.sum(pj, axis=1, keepdims=True)
    gsum = jnp.sum(gj, axis=1, keepdims=True)

    c0 = (jnp.sum(o * wv_ref[:, :DP], axis=1, keepdims=True)
          + bv_ref[0, 0] + (sp - w * gsum) * (1.0 / A))
    v = c0 + w * gj * (1.0 / A)

    xv_ref[...] = jnp.tile(v, (1, A))
    wmb_ref[...] = jnp.broadcast_to(w, (v.shape[0], A * A))


def kernel(x, policies, actions, edge_index, W1, b1, W2, b2, Wfc, Wattn, Wv, bv):
    x3 = x.reshape(B, A, DIN)
    pi3 = policies.reshape(B, A, ACT)
    ac3 = actions.reshape(B, A, ACT)

    row_blk = lambda i: (i, 0, 0)
    whole = lambda i: (0, 0)

    xv_flat, wmb_flat = pl.pallas_call(
        _critic_kernel,
        grid=(NB,),
        in_specs=[
            pl.BlockSpec((TB, A, DIN), row_blk),
            pl.BlockSpec((TB, A, ACT), row_blk),
            pl.BlockSpec((TB, A, ACT), row_blk),
            pl.BlockSpec((H1, DIN), whole),
            pl.BlockSpec((1, H1), whole),
            pl.BlockSpec((DP, H1), whole),
            pl.BlockSpec((1, DP), whole),
            pl.BlockSpec((WOUT, DP), whole),
            pl.BlockSpec((1, 2 * WOUT), whole),
            pl.BlockSpec((1, DP + ACT), whole),
            pl.BlockSpec((1, 1), whole),
        ],
        out_specs=(
            pl.BlockSpec((TB, A * A), lambda i: (i, 0)),
            pl.BlockSpec((TB, A * A), lambda i: (i, 0)),
        ),
        out_shape=(
            jax.ShapeDtypeStruct((B, A * A), jnp.float32),
            jax.ShapeDtypeStruct((B, A * A), jnp.float32),
        ),
    )(x3, pi3, ac3, W1, b1.reshape(1, H1), W2, b2.reshape(1, DP), Wfc,
      Wattn, Wv, bv.reshape(1, 1))

    xv = xv_flat.reshape(N, A, 1)
    w_mb = wmb_flat.reshape(N, A, 1)
    return xv, w_mb


# speedup vs baseline: 1.0292x; 1.0292x over previous
import jax
import jax.numpy as jnp
from jax import lax
from jax.experimental import pallas as pl

B = 625
A = 16
N = B * A
DIN = 128
H1 = 64
DP = 64
WOUT = 64
ACT = 8

TB = 256
NB = (B + TB - 1) // TB

_DN11 = (((1,), (1,)), ((), ()))


def _critic_kernel(x3_ref, pi3_ref, ac3_ref,
                   w1_ref, b1_ref, w2_ref, b2_ref, wfc_ref,
                   wat_ref, wv_ref, bv_ref,
                   xv_ref, wmb_ref):
    f32 = jnp.float32
    xm = jnp.sum(x3_ref[...], axis=1) * (1.0 / A)

    h = jnp.maximum(
        lax.dot_general(xm, w1_ref[...], _DN11, preferred_element_type=f32)
        + b1_ref[...], 0.0)
    o = (lax.dot_general(h, w2_ref[...], _DN11, preferred_element_type=f32)
         + b2_ref[...])
    zl = lax.dot_general(o, wfc_ref[...], _DN11, preferred_element_type=f32)

    wa = wat_ref[:, :WOUT] + wat_ref[:, WOUT:]
    e = jnp.sum(zl * wa, axis=1, keepdims=True)
    e = jnp.where(e >= 0.0, e, 0.01 * e)
    w = jax.nn.sigmoid(e)

    wz3 = wv_ref[:, DP:].reshape(1, 1, ACT)
    pi3 = pi3_ref[...]
    d3 = pi3 - ac3_ref[...]
    gj = jnp.sum(d3 * wz3, axis=2)
    pj = jnp.sum(pi3 * wz3, axis=2)
    sp = jnp.sum(pj, axis=1, keepdims=True)
    gsum = jnp.sum(gj, axis=1, keepdims=True)

    c0 = (jnp.sum(o * wv_ref[:, :DP], axis=1, keepdims=True)
          + bv_ref[0, 0] + (sp - w * gsum) * (1.0 / A))
    v = c0 + w * gj * (1.0 / A)

    xv_ref[...] = jnp.tile(v, (1, A))
    wmb_ref[...] = jnp.broadcast_to(w, (v.shape[0], A * A))


def kernel(x, policies, actions, edge_index, W1, b1, W2, b2, Wfc, Wattn, Wv, bv):
    x3 = x.reshape(B, A, DIN)
    pi3 = policies.reshape(B, A, ACT)
    ac3 = actions.reshape(B, A, ACT)

    row_blk = lambda i: (i, 0, 0)
    whole = lambda i: (0, 0)

    xv_flat, wmb_flat = pl.pallas_call(
        _critic_kernel,
        grid=(NB,),
        in_specs=[
            pl.BlockSpec((TB, A, DIN), row_blk),
            pl.BlockSpec((TB, A, ACT), row_blk),
            pl.BlockSpec((TB, A, ACT), row_blk),
            pl.BlockSpec((H1, DIN), whole),
            pl.BlockSpec((1, H1), whole),
            pl.BlockSpec((DP, H1), whole),
            pl.BlockSpec((1, DP), whole),
            pl.BlockSpec((WOUT, DP), whole),
            pl.BlockSpec((1, 2 * WOUT), whole),
            pl.BlockSpec((1, DP + ACT), whole),
            pl.BlockSpec((1, 1), whole),
        ],
        out_specs=(
            pl.BlockSpec((TB, A * A), lambda i: (i, 0)),
            pl.BlockSpec((TB, A * A), lambda i: (i, 0)),
        ),
        out_shape=(
            jax.ShapeDtypeStruct((B, A * A), jnp.float32),
            jax.ShapeDtypeStruct((B, A * A), jnp.float32),
        ),
    )(x3, pi3, ac3, W1, b1.reshape(1, H1), W2, b2.reshape(1, DP), Wfc,
      Wattn, Wv, bv.reshape(1, 1))

    xv = xv_flat.reshape(N, A, 1)
    w_mb = wmb_flat.reshape(N, A, 1)
    return xv, w_mb


# speedup vs baseline: 1.0520x; 1.0221x over previous
import jax
import jax.numpy as jnp
from jax import lax
from jax.experimental import pallas as pl

B = 625
A = 16
N = B * A
DIN = 128
H1 = 64
DP = 64
WOUT = 64
ACT = 8

TB = 320
NB = (B + TB - 1) // TB

_DN11 = (((1,), (1,)), ((), ()))


def _critic_kernel(x3_ref, pi3_ref, ac3_ref,
                   w1_ref, b1_ref, w2_ref, b2_ref, wfc_ref,
                   wat_ref, wv_ref, bv_ref,
                   xv_ref, wmb_ref):
    f32 = jnp.float32
    xm = jnp.sum(x3_ref[...], axis=1) * (1.0 / A)

    h = jnp.maximum(
        lax.dot_general(xm, w1_ref[...], _DN11, preferred_element_type=f32)
        + b1_ref[...], 0.0)
    o = (lax.dot_general(h, w2_ref[...], _DN11, preferred_element_type=f32)
         + b2_ref[...])
    zl = lax.dot_general(o, wfc_ref[...], _DN11, preferred_element_type=f32)

    wa = wat_ref[:, :WOUT] + wat_ref[:, WOUT:]
    e = jnp.sum(zl * wa, axis=1, keepdims=True)
    e = jnp.where(e >= 0.0, e, 0.01 * e)
    w = jax.nn.sigmoid(e)

    wz3 = wv_ref[:, DP:].reshape(1, 1, ACT)
    pi3 = pi3_ref[...]
    d3 = pi3 - ac3_ref[...]
    gj = jnp.sum(d3 * wz3, axis=2)
    pj = jnp.sum(pi3 * wz3, axis=2)
    sp = jnp.sum(pj, axis=1, keepdims=True)
    gsum = jnp.sum(gj, axis=1, keepdims=True)

    c0 = (jnp.sum(o * wv_ref[:, :DP], axis=1, keepdims=True)
          + bv_ref[0, 0] + (sp - w * gsum) * (1.0 / A))
    v = c0 + w * gj * (1.0 / A)

    xv_ref[...] = jnp.tile(v, (1, A))
    wmb_ref[...] = jnp.broadcast_to(w, (v.shape[0], A * A))


def kernel(x, policies, actions, edge_index, W1, b1, W2, b2, Wfc, Wattn, Wv, bv):
    x3 = x.reshape(B, A, DIN)
    pi3 = policies.reshape(B, A, ACT)
    ac3 = actions.reshape(B, A, ACT)

    row_blk = lambda i: (i, 0, 0)
    whole = lambda i: (0, 0)

    xv_flat, wmb_flat = pl.pallas_call(
        _critic_kernel,
        grid=(NB,),
        in_specs=[
            pl.BlockSpec((TB, A, DIN), row_blk),
            pl.BlockSpec((TB, A, ACT), row_blk),
            pl.BlockSpec((TB, A, ACT), row_blk),
            pl.BlockSpec((H1, DIN), whole),
            pl.BlockSpec((1, H1), whole),
            pl.BlockSpec((DP, H1), whole),
            pl.BlockSpec((1, DP), whole),
            pl.BlockSpec((WOUT, DP), whole),
            pl.BlockSpec((1, 2 * WOUT), whole),
            pl.BlockSpec((1, DP + ACT), whole),
            pl.BlockSpec((1, 1), whole),
        ],
        out_specs=(
            pl.BlockSpec((TB, A * A), lambda i: (i, 0)),
            pl.BlockSpec((TB, A * A), lambda i: (i, 0)),
        ),
        out_shape=(
            jax.ShapeDtypeStruct((B, A * A), jnp.float32),
            jax.ShapeDtypeStruct((B, A * A), jnp.float32),
        ),
    )(x3, pi3, ac3, W1, b1.reshape(1, H1), W2, b2.reshape(1, DP), Wfc,
      Wattn, Wv, bv.reshape(1, 1))

    xv = xv_flat.reshape(N, A, 1)
    w_mb = wmb_flat.reshape(N, A, 1)
    return xv, w_mb
